# baseline (speedup 1.0000x reference)
import functools

import jax
import jax.numpy as jnp
from jax import lax
from jax.experimental import pallas as pl
from jax.experimental.pallas import tpu as pltpu
from jax.experimental.pallas import tpu_sc as plsc

_ROWS = 64
_COLS = 32768
_J = _COLS // 2
_NW = 32
_ROWS_PER_W = _ROWS // _NW
_SHIFTS = (23, 15, 7, 0)
_NBITS = (8, 8, 8, 7)
_HISTW = 16 * 256


def _sc_threshold_body(
    scores_hbm, out_hbm, data_v, hist_v, comp_v, csums_v, excl_v, res_v, dma_sem
):
    cid = lax.axis_index("c")
    sid = lax.axis_index("s")
    wid = sid * 2 + cid

    pltpu.sync_copy(scores_hbm.at[pl.ds(_ROWS_PER_W * wid, 1)], data_v.at[0:1])
    row1 = pltpu.async_copy(
        scores_hbm.at[pl.ds(_ROWS_PER_W * wid + 1, 1)], data_v.at[1:2], dma_sem
    )

    lane = lax.iota(jnp.int32, 16)
    idx_base = lane * 256
    ones = jnp.ones((16,), jnp.int32)
    zeros16 = jnp.zeros((16,), jnp.int32)

    def splat(x):
        return jnp.full((16,), 1, jnp.int32) * x

    def clear_body(i, c):
        hist_v[pl.ds(pl.multiple_of(i * 16, 16), 16)] = zeros16
        return c

    lax.fori_loop(0, _HISTW // 16, clear_body, 0)

    def scan_hist(nvreg, jrem_v):
        @plsc.parallel_loop(0, nvreg, 1, unroll=2)
        def _(v):
            st = pl.multiple_of(v * 16, 16)
            p = st + lane
            s = zeros16
            for l in range(16):
                idx = l * 256 + ((p + l) & jnp.int32(255))
                s = s + plsc.load_gather(hist_v, [idx])
                plsc.store_scatter(hist_v, [idx], zeros16)
            cl = plsc.cumsum(s)
            csums_v[pl.ds(st, 16)] = cl

        t = plsc.load_gather(csums_v, [lane * 16 + 15])
        ct = plsc.cumsum(t)
        excl_v[...] = ct - t

        @plsc.parallel_loop(0, nvreg, 1, unroll=2, carry=(zeros16, zeros16))
        def stageb(v, carry):
            pcnt_v, cb_v = carry
            e = plsc.load_gather(excl_v, [splat(v)])
            cg = csums_v[pl.ds(v * 16, 16)] + e
            m = cg <= jrem_v
            pcnt_v = pcnt_v + plsc.all_reduce_population_count(m)
            cb_v = jnp.maximum(cb_v, jnp.where(m, cg, 0))
            return (pcnt_v, cb_v)

        pcnt_v, cb_v = stageb
        return jnp.max(pcnt_v), jnp.max(cb_v)

    thr_vec = jnp.zeros((16,), jnp.int32)
    for r in range(_ROWS_PER_W):
        if r == 1:
            row1.wait()
        prefix = jnp.int32(0)
        jrem_v = jnp.full((16,), _J, jnp.int32)
        s1_v = zeros16
        n2 = jnp.int32(0)
        for pi in range(4):
            shift = _SHIFTS[pi]
            nbins = 1 << _NBITS[pi]
            binmask = jnp.int32(nbins - 1)

            if pi == 0:

                @plsc.parallel_loop(0, _COLS // 16, 1, unroll=8)
                def _(i):
                    st = pl.multiple_of(i * 16, 16)
                    v = data_v[r, pl.ds(st, 16)]
                    b = lax.bitcast_convert_type(jnp.abs(v), jnp.int32)
                    bucket = lax.shift_right_logical(b, _SHIFTS[0])
                    rot = (bucket + lane) & jnp.int32(255)
                    plsc.addupdate_scatter(hist_v, [idx_base + rot], ones)

            elif pi == 1:
                pref_v = splat(prefix)

                @plsc.parallel_loop(0, _COLS // 16, 1, unroll=8, carry=zeros16)
                def compact(i, cnt_v, _pv=pref_v):
                    st = pl.multiple_of(i * 16, 16)
                    v = data_v[r, pl.ds(st, 16)]
                    b = lax.bitcast_convert_type(jnp.abs(v), jnp.int32)
                    m = lax.shift_right_logical(b, _SHIFTS[0]) == _pv
                    bucket = lax.shift_right_logical(b, _SHIFTS[1]) & jnp.int32(255)
                    rot = (bucket + lane) & jnp.int32(255)
                    plsc.addupdate_scatter(hist_v, [idx_base + rot], ones, mask=m)
                    plsc.store_scatter(comp_v, [lane * 2049 + cnt_v], b, mask=m)
                    return cnt_v + jnp.where(m, 1, 0).astype(jnp.int32)

                s1_v = compact
                n2 = jnp.max(s1_v)

            else:
                match_shift = _SHIFTS[pi - 1]
                pref_v = splat(prefix)

                @plsc.parallel_loop(0, n2, 1, unroll=4)
                def _(i, _ms=match_shift, _sh=shift, _bm=binmask, _pv=pref_v):
                    c = plsc.load_gather(comp_v, [lane * 2049 + i])
                    inr = splat(i) < s1_v
                    m = (lax.shift_right_logical(c, _ms) == _pv) & inr
                    bucket = lax.shift_right_logical(c, _sh) & _bm
                    rot = (bucket + lane) & jnp.int32(255)
                    plsc.addupdate_scatter(hist_v, [idx_base + rot], ones, mask=m)

            bucket_p, cum_below = scan_hist(nbins // 16, jrem_v)
            jrem_v = jrem_v - cum_below
            prefix = lax.shift_left(prefix, _NBITS[pi]) | bucket_p

        thr_vec = jnp.where(lane == r, prefix, thr_vec)

    res_v[0, :] = thr_vec
    pltpu.sync_copy(res_v, out_hbm.at[pl.ds(wid, 1)])


_sc_thresholds = functools.partial(
    pl.kernel,
    mesh=plsc.VectorSubcoreMesh(core_axis_name="c", subcore_axis_name="s"),
    compiler_params=pltpu.CompilerParams(needs_layout_passes=False),
    out_type=jax.ShapeDtypeStruct((_NW, 16), jnp.int32),
    scratch_types=[
        pltpu.VMEM((_ROWS_PER_W, _COLS), jnp.float32),
        pltpu.VMEM((_HISTW,), jnp.int32),
        pltpu.VMEM((16 * 2049,), jnp.int32),
        pltpu.VMEM((256,), jnp.int32),
        pltpu.VMEM((16,), jnp.int32),
        pltpu.VMEM((1, 16), jnp.int32),
        pltpu.SemaphoreType.DMA,
    ],
)(_sc_threshold_body)


_BLOCK_ROWS = 8


def _apply_kernel(x_ref, s_ref, t_ref, out_ref, mask_ref):
    bits = lax.bitcast_convert_type(jnp.abs(s_ref[...]), jnp.int32)
    thr = t_ref[:, 0:1]
    mask = (bits >= thr).astype(jnp.float32)
    mask_ref[...] = mask
    out_ref[...] = x_ref[...] * mask


@jax.jit
def kernel(x, scores):
    thr2d = _sc_thresholds(scores)
    thr = thr2d[:, :_ROWS_PER_W].reshape(_ROWS)
    thr_b = jnp.broadcast_to(thr[:, None], (_ROWS, 128))

    spec = pl.BlockSpec((_BLOCK_ROWS, _COLS), lambda i: (i, 0))
    tspec = pl.BlockSpec((_BLOCK_ROWS, 128), lambda i: (i, 0))
    out, mask = pl.pallas_call(
        _apply_kernel,
        grid=(_ROWS // _BLOCK_ROWS,),
        in_specs=[spec, spec, tspec],
        out_specs=[spec, spec],
        out_shape=[
            jax.ShapeDtypeStruct((_ROWS, _COLS), jnp.float32),
            jax.ShapeDtypeStruct((_ROWS, _COLS), jnp.float32),
        ],
    )(x, scores, thr_b)
    return (out, mask)


# speedup vs baseline: 1.0058x; 1.0058x over previous
import functools

import jax
import jax.numpy as jnp
from jax import lax
from jax.experimental import pallas as pl
from jax.experimental.pallas import tpu as pltpu
from jax.experimental.pallas import tpu_sc as plsc

_ROWS = 64
_COLS = 32768
_J = _COLS // 2
_NW = 32
_ROWS_PER_W = _ROWS // _NW
_SHIFTS = (23, 15, 7, 0)
_NBITS = (8, 8, 8, 7)
_HISTW = 16 * 256


def _sc_threshold_body(
    scores_hbm, out_hbm, data_v, hist_v, comp_v, csums_v, excl_v, res_v
):
    cid = lax.axis_index("c")
    sid = lax.axis_index("s")
    wid = sid * 2 + cid

    pltpu.sync_copy(scores_hbm.at[pl.ds(_ROWS_PER_W * wid, _ROWS_PER_W)], data_v)

    lane = lax.iota(jnp.int32, 16)
    idx_base = lane * 256
    ones = jnp.ones((16,), jnp.int32)
    zeros16 = jnp.zeros((16,), jnp.int32)

    def splat(x):
        return jnp.full((16,), 1, jnp.int32) * x

    def clear_body(i, c):
        hist_v[pl.ds(pl.multiple_of(i * 16, 16), 16)] = zeros16
        return c

    lax.fori_loop(0, _HISTW // 16, clear_body, 0)

    def scan_hist(nvreg, jrem_v):
        @plsc.parallel_loop(0, nvreg, 1, unroll=2)
        def _(v):
            st = pl.multiple_of(v * 16, 16)
            p = st + lane
            s = zeros16
            for l in range(16):
                idx = l * 256 + ((p + l) & jnp.int32(255))
                s = s + plsc.load_gather(hist_v, [idx])
                plsc.store_scatter(hist_v, [idx], zeros16)
            cl = plsc.cumsum(s)
            csums_v[pl.ds(st, 16)] = cl

        t = plsc.load_gather(csums_v, [lane * 16 + 15])
        ct = plsc.cumsum(t)
        excl_v[...] = ct - t

        @plsc.parallel_loop(0, nvreg, 1, unroll=2, carry=(zeros16, zeros16))
        def stageb(v, carry):
            pcnt_v, cb_v = carry
            e = plsc.load_gather(excl_v, [splat(v)])
            cg = csums_v[pl.ds(v * 16, 16)] + e
            m = cg <= jrem_v
            pcnt_v = pcnt_v + plsc.all_reduce_population_count(m)
            cb_v = jnp.maximum(cb_v, jnp.where(m, cg, 0))
            return (pcnt_v, cb_v)

        pcnt_v, cb_v = stageb
        return jnp.max(pcnt_v), jnp.max(cb_v)

    thr_vec = jnp.zeros((16,), jnp.int32)
    for r in range(_ROWS_PER_W):
        prefix = jnp.int32(0)
        jrem_v = jnp.full((16,), _J, jnp.int32)
        s1_v = zeros16
        n2 = jnp.int32(0)
        for pi in range(4):
            shift = _SHIFTS[pi]
            nbins = 1 << _NBITS[pi]
            binmask = jnp.int32(nbins - 1)

            if pi == 0:

                @plsc.parallel_loop(0, _COLS // 16, 1, unroll=8)
                def _(i):
                    st = pl.multiple_of(i * 16, 16)
                    v = data_v[r, pl.ds(st, 16)]
                    b = lax.bitcast_convert_type(jnp.abs(v), jnp.int32)
                    bucket = lax.shift_right_logical(b, _SHIFTS[0])
                    rot = (bucket + lane) & jnp.int32(255)
                    plsc.addupdate_scatter(hist_v, [idx_base + rot], ones)

            elif pi == 1:
                pref_v = splat(prefix)

                @plsc.parallel_loop(0, _COLS // 16, 1, unroll=8, carry=zeros16)
                def compact(i, cnt_v, _pv=pref_v):
                    st = pl.multiple_of(i * 16, 16)
                    v = data_v[r, pl.ds(st, 16)]
                    b = lax.bitcast_convert_type(jnp.abs(v), jnp.int32)
                    m = lax.shift_right_logical(b, _SHIFTS[0]) == _pv
                    bucket = lax.shift_right_logical(b, _SHIFTS[1]) & jnp.int32(255)
                    rot = (bucket + lane) & jnp.int32(255)
                    plsc.addupdate_scatter(hist_v, [idx_base + rot], ones, mask=m)
                    plsc.store_scatter(comp_v, [lane * 2049 + cnt_v], b, mask=m)
                    return cnt_v + jnp.where(m, 1, 0).astype(jnp.int32)

                s1_v = compact
                n2 = jnp.max(s1_v)

            else:
                match_shift = _SHIFTS[pi - 1]
                pref_v = splat(prefix)

                @plsc.parallel_loop(0, n2, 1, unroll=4)
                def _(i, _ms=match_shift, _sh=shift, _bm=binmask, _pv=pref_v):
                    c = plsc.load_gather(comp_v, [lane * 2049 + i])
                    inr = splat(i) < s1_v
                    m = (lax.shift_right_logical(c, _ms) == _pv) & inr
                    bucket = lax.shift_right_logical(c, _sh) & _bm
                    rot = (bucket + lane) & jnp.int32(255)
                    plsc.addupdate_scatter(hist_v, [idx_base + rot], ones, mask=m)

            bucket_p, cum_below = scan_hist(nbins // 16, jrem_v)
            jrem_v = jrem_v - cum_below
            prefix = lax.shift_left(prefix, _NBITS[pi]) | bucket_p

        thr_vec = jnp.where(lane == r, prefix, thr_vec)

    res_v[0, :] = thr_vec
    pltpu.sync_copy(res_v, out_hbm.at[pl.ds(wid, 1)])


_sc_thresholds = functools.partial(
    pl.kernel,
    mesh=plsc.VectorSubcoreMesh(core_axis_name="c", subcore_axis_name="s"),
    compiler_params=pltpu.CompilerParams(needs_layout_passes=False),
    out_type=jax.ShapeDtypeStruct((_NW, 16), jnp.int32),
    scratch_types=[
        pltpu.VMEM((_ROWS_PER_W, _COLS), jnp.float32),
        pltpu.VMEM((_HISTW,), jnp.int32),
        pltpu.VMEM((16 * 2049,), jnp.int32),
        pltpu.VMEM((256,), jnp.int32),
        pltpu.VMEM((16,), jnp.int32),
        pltpu.VMEM((1, 16), jnp.int32),
    ],
)(_sc_threshold_body)


_BLOCK_ROWS = 8


def _apply_kernel(x_ref, s_ref, t_ref, out_ref, mask_ref):
    bits = lax.bitcast_convert_type(jnp.abs(s_ref[...]), jnp.int32)
    thr = t_ref[:, 0:1]
    mask = (bits >= thr).astype(jnp.float32)
    mask_ref[...] = mask
    out_ref[...] = x_ref[...] * mask


@jax.jit
def kernel(x, scores):
    thr2d = _sc_thresholds(scores)
    thr = thr2d[:, :_ROWS_PER_W].reshape(_ROWS)
    thr_b = jnp.broadcast_to(thr[:, None], (_ROWS, 128))

    spec = pl.BlockSpec((_BLOCK_ROWS, _COLS), lambda i: (i, 0))
    tspec = pl.BlockSpec((_BLOCK_ROWS, 128), lambda i: (i, 0))
    out, mask = pl.pallas_call(
        _apply_kernel,
        grid=(_ROWS // _BLOCK_ROWS,),
        in_specs=[spec, spec, tspec],
        out_specs=[spec, spec],
        out_shape=[
            jax.ShapeDtypeStruct((_ROWS, _COLS), jnp.float32),
            jax.ShapeDtypeStruct((_ROWS, _COLS), jnp.float32),
        ],
    )(x, scores, thr_b)
    return (out, mask)
